# Initial kernel scaffold; baseline (speedup 1.0000x reference)
import jax
import jax.numpy as jnp
from jax.experimental import pallas as pl


def kernel(pc, params):
    raise NotImplementedError("write your pallas kernel here")


# speedup vs baseline: 3.4779x; 3.4779x over previous
import functools

import jax
import jax.numpy as jnp
from jax.experimental import pallas as pl

_HI = jax.lax.Precision.HIGHEST


def _fps_kernel(x_ref, y_ref, z_ref, cx_ref, cy_ref, cz_ref, *, npoint):
    x = x_ref[...]
    y = y_ref[...]
    z = z_ref[...]
    B, N = x.shape
    iota_n = jax.lax.broadcasted_iota(jnp.int32, (B, N), 1)
    iota_s = jax.lax.broadcasted_iota(jnp.int32, (B, npoint), 1)
    dists0 = jnp.full((B, N), 1e10, dtype=jnp.float32)
    lastoh0 = (iota_n == 0).astype(jnp.float32)
    acc0 = jnp.zeros((B, npoint), dtype=jnp.float32)

    def body(i, carry):
        dists, lastoh, cx, cy, cz = carry
        px = jnp.sum(x * lastoh, axis=1, keepdims=True)
        py = jnp.sum(y * lastoh, axis=1, keepdims=True)
        pz = jnp.sum(z * lastoh, axis=1, keepdims=True)
        cx = jnp.where(iota_s == i, px, cx)
        cy = jnp.where(iota_s == i, py, cy)
        cz = jnp.where(iota_s == i, pz, cz)
        dx = x - px
        dy = y - py
        dz = z - pz
        d = dx * dx + dy * dy + dz * dz
        dists = jnp.minimum(dists, d)
        m = jnp.max(dists, axis=1, keepdims=True)
        cand = jnp.where(dists == m, iota_n, N)
        nmin = jnp.min(cand, axis=1, keepdims=True)
        lastoh = (iota_n == nmin).astype(jnp.float32)
        return dists, lastoh, cx, cy, cz

    _, _, cx, cy, cz = jax.lax.fori_loop(
        0, npoint, body, (dists0, lastoh0, acc0, acc0, acc0))
    cx_ref[...] = cx
    cy_ref[...] = cy
    cz_ref[...] = cz


def _fps(x, y, z, npoint):
    B, N = x.shape
    out = jax.ShapeDtypeStruct((B, npoint), jnp.float32)
    return pl.pallas_call(
        functools.partial(_fps_kernel, npoint=npoint),
        out_shape=(out, out, out),
    )(x, y, z)


def _proj_kernel(x_ref, w_ref, y_ref):
    y_ref[0] = jnp.dot(x_ref[0], w_ref[...], precision=_HI,
                       preferred_element_type=jnp.float32)


def _proj(X, W1T):
    B, N, Cin = X.shape
    C1 = W1T.shape[1]
    return pl.pallas_call(
        _proj_kernel,
        grid=(B,),
        in_specs=[
            pl.BlockSpec((1, N, Cin), lambda b: (b, 0, 0)),
            pl.BlockSpec((Cin, C1), lambda b: (0, 0)),
        ],
        out_specs=pl.BlockSpec((1, N, C1), lambda b: (b, 0, 0)),
        out_shape=jax.ShapeDtypeStruct((B, N, C1), jnp.float32),
    )(X, W1T)


def _l1_kernel(x_ref, xyzT_ref, cen_ref, w_ref, b_ref, out_ref,
               stats_ref, *, r2, K, Sb, N):
    b = pl.program_id(0)
    j = pl.program_id(1)

    @pl.when(jnp.logical_and(b == 0, j == 0))
    def _():
        stats_ref[...] = jnp.zeros_like(stats_ref)

    c = cen_ref[0]
    xr = xyzT_ref[0, 0:1, :]
    yr = xyzT_ref[0, 1:2, :]
    zr = xyzT_ref[0, 2:3, :]
    dx = c[:, 0:1] - xr
    dy = c[:, 1:2] - yr
    dz = c[:, 2:3] - zr
    sqd = dx * dx + dy * dy + dz * dz
    mask = sqd <= r2
    mask_f = mask.astype(jnp.float32)
    im = jax.lax.broadcasted_iota(jnp.int32, (N, N), 0)
    in_ = jax.lax.broadcasted_iota(jnp.int32, (N, N), 1)
    tri = (im <= in_).astype(jnp.float32)
    rank = jnp.dot(mask_f, tri,
                   preferred_element_type=jnp.float32).astype(jnp.int32)
    count = jnp.sum(mask_f, axis=1).astype(jnp.int32).reshape(Sb, 1, 1)
    rank3 = rank.reshape(Sb, 1, N)
    mask3 = mask.reshape(Sb, 1, N)
    kp1 = jax.lax.broadcasted_iota(jnp.int32, (1, K, 1), 1) + 1
    sel = (rank3 == kp1) | ((kp1 > count) & (rank3 == 1))
    onehot = (sel & mask3).astype(jnp.float32).reshape(Sb * K, N)
    gx = jnp.dot(onehot, x_ref[0], precision=_HI,
                 preferred_element_type=jnp.float32)
    Cin = gx.shape[1]
    inp = gx.reshape(Sb, K, Cin) - c.reshape(Sb, 1, Cin)
    C1 = w_ref.shape[1]
    h = jnp.dot(inp.reshape(Sb * K, Cin), w_ref[...],
                preferred_element_type=jnp.float32)
    h = h.reshape(Sb, K, C1) + b_ref[...].reshape(1, 1, C1)
    out_ref[0] = h
    stats_ref[0:1, :] += jnp.sum(h, axis=(0, 1)).reshape(1, C1)
    stats_ref[1:2, :] += jnp.sum(h * h, axis=(0, 1)).reshape(1, C1)


def _l1(X, xyzT, centersPad, W1T, b1, r2, K, Sb):
    B, N, Cin = X.shape
    S = centersPad.shape[1]
    C1 = W1T.shape[1]
    grid = (B, S // Sb)
    out, stats = pl.pallas_call(
        functools.partial(_l1_kernel, r2=r2, K=K, Sb=Sb, N=N),
        grid=grid,
        in_specs=[
            pl.BlockSpec((1, N, Cin), lambda b, j: (b, 0, 0)),
            pl.BlockSpec((1, 3, N), lambda b, j: (b, 0, 0)),
            pl.BlockSpec((1, Sb, Cin), lambda b, j: (b, j, 0)),
            pl.BlockSpec((Cin, C1), lambda b, j: (0, 0)),
            pl.BlockSpec((1, C1), lambda b, j: (0, 0)),
        ],
        out_specs=(
            pl.BlockSpec((1, Sb, K, C1), lambda b, j: (b, j, 0, 0)),
            pl.BlockSpec((2, C1), lambda b, j: (0, 0)),
        ),
        out_shape=(
            jax.ShapeDtypeStruct((B, S, K, C1), jnp.float32),
            jax.ShapeDtypeStruct((2, C1), jnp.float32),
        ),
    )(X, xyzT, centersPad, W1T, b1.reshape(1, -1))
    return out, stats


def _layer_kernel(x_ref, m_ref, v_ref, g_ref, be_ref, w_ref, b_ref,
                  out_ref, stats_ref):
    b = pl.program_id(0)
    j = pl.program_id(1)

    @pl.when(jnp.logical_and(b == 0, j == 0))
    def _():
        stats_ref[...] = jnp.zeros_like(stats_ref)

    x = x_ref[0]
    Sb, K, Cp = x.shape
    xn = ((x - m_ref[...].reshape(1, 1, Cp))
          / jnp.sqrt(v_ref[...].reshape(1, 1, Cp) + 1e-5)
          * g_ref[...].reshape(1, 1, Cp) + be_ref[...].reshape(1, 1, Cp))
    h = jax.nn.relu(xn)
    Cn = w_ref.shape[1]
    o = jnp.dot(h.reshape(Sb * K, Cp), w_ref[...],
                preferred_element_type=jnp.float32)
    o = o.reshape(Sb, K, Cn) + b_ref[...].reshape(1, 1, Cn)
    out_ref[0] = o
    stats_ref[0:1, :] += jnp.sum(o, axis=(0, 1)).reshape(1, Cn)
    stats_ref[1:2, :] += jnp.sum(o * o, axis=(0, 1)).reshape(1, Cn)


def _layer(x, m, v, g, be, WT, b, Sb):
    B, S, K, Cp = x.shape
    Cn = WT.shape[1]
    grid = (B, S // Sb)
    out, stats = pl.pallas_call(
        _layer_kernel,
        grid=grid,
        in_specs=[
            pl.BlockSpec((1, Sb, K, Cp), lambda bb, j: (bb, j, 0, 0)),
            pl.BlockSpec((1, Cp), lambda bb, j: (0, 0)),
            pl.BlockSpec((1, Cp), lambda bb, j: (0, 0)),
            pl.BlockSpec((1, Cp), lambda bb, j: (0, 0)),
            pl.BlockSpec((1, Cp), lambda bb, j: (0, 0)),
            pl.BlockSpec((Cp, Cn), lambda bb, j: (0, 0)),
            pl.BlockSpec((1, Cn), lambda bb, j: (0, 0)),
        ],
        out_specs=(
            pl.BlockSpec((1, Sb, K, Cn), lambda bb, j: (bb, j, 0, 0)),
            pl.BlockSpec((2, Cn), lambda bb, j: (0, 0)),
        ),
        out_shape=(
            jax.ShapeDtypeStruct((B, S, K, Cn), jnp.float32),
            jax.ShapeDtypeStruct((2, Cn), jnp.float32),
        ),
    )(x, m.reshape(1, -1), v.reshape(1, -1), g.reshape(1, -1),
      be.reshape(1, -1), WT, b.reshape(1, -1))
    return out, stats


def _final_kernel(x_ref, m_ref, v_ref, g_ref, be_ref, out_ref):
    x = x_ref[0]
    C = x.shape[2]
    xn = ((x - m_ref[...].reshape(1, 1, C))
          / jnp.sqrt(v_ref[...].reshape(1, 1, C) + 1e-5)
          * g_ref[...].reshape(1, 1, C) + be_ref[...].reshape(1, 1, C))
    h = jax.nn.relu(xn)
    out_ref[0] = jnp.max(h, axis=1)


def _final(x, m, v, g, be, Sb):
    B, S, K, C = x.shape
    grid = (B, S // Sb)
    return pl.pallas_call(
        _final_kernel,
        grid=grid,
        in_specs=[
            pl.BlockSpec((1, Sb, K, C), lambda bb, j: (bb, j, 0, 0)),
            pl.BlockSpec((1, C), lambda bb, j: (0, 0)),
            pl.BlockSpec((1, C), lambda bb, j: (0, 0)),
            pl.BlockSpec((1, C), lambda bb, j: (0, 0)),
            pl.BlockSpec((1, C), lambda bb, j: (0, 0)),
        ],
        out_specs=pl.BlockSpec((1, Sb, C), lambda bb, j: (bb, j, 0)),
        out_shape=jax.ShapeDtypeStruct((B, S, C), jnp.float32),
    )(x, m.reshape(1, -1), v.reshape(1, -1), g.reshape(1, -1),
      be.reshape(1, -1))


def _fc_kernel(x_ref, w1_ref, b1_ref, g1_ref, be1_ref,
               w2_ref, b2_ref, g2_ref, be2_ref,
               w3_ref, b3_ref, out_ref):
    x = x_ref[...]

    def bn_relu(h, g, be):
        m = jnp.mean(h, axis=0, keepdims=True)
        v = jnp.mean((h - m) * (h - m), axis=0, keepdims=True)
        return jax.nn.relu((h - m) / jnp.sqrt(v + 1e-5) * g + be)

    h = jnp.dot(x, w1_ref[...],
                preferred_element_type=jnp.float32) + b1_ref[...]
    h = bn_relu(h, g1_ref[...], be1_ref[...])
    h = jnp.dot(h, w2_ref[...],
                preferred_element_type=jnp.float32) + b2_ref[...]
    h = bn_relu(h, g2_ref[...], be2_ref[...])
    h = jnp.dot(h, w3_ref[...],
                preferred_element_type=jnp.float32) + b3_ref[...]
    out_ref[...] = h


def _fc(x, fc_params):
    (W1, b1, g1, be1), (W2, b2, g2, be2), (W3, b3, _, _) = fc_params
    B = x.shape[0]
    return pl.pallas_call(
        _fc_kernel,
        out_shape=jax.ShapeDtypeStruct((B, W3.shape[0]), jnp.float32),
    )(x, W1.T, b1.reshape(1, -1), g1.reshape(1, -1), be1.reshape(1, -1),
      W2.T, b2.reshape(1, -1), g2.reshape(1, -1), be2.reshape(1, -1),
      W3.T, b3.reshape(1, -1))


def _var_kernel(x_ref, m_ref, o_ref):
    b = pl.program_id(0)
    j = pl.program_id(1)

    @pl.when(jnp.logical_and(b == 0, j == 0))
    def _():
        o_ref[...] = jnp.zeros_like(o_ref)

    x = x_ref[0]
    C = x.shape[2]
    dv = x - m_ref[...].reshape(1, 1, C)
    o_ref[0:1, :] += jnp.sum(dv * dv, axis=(0, 1)).reshape(1, C)


def _var(x, m, Sb):
    B, S, K, C = x.shape
    grid = (B, S // Sb)
    out = pl.pallas_call(
        _var_kernel,
        grid=grid,
        in_specs=[
            pl.BlockSpec((1, Sb, K, C), lambda bb, j: (bb, j, 0, 0)),
            pl.BlockSpec((1, C), lambda bb, j: (0, 0)),
        ],
        out_specs=pl.BlockSpec((1, C), lambda bb, j: (0, 0)),
        out_shape=jax.ShapeDtypeStruct((1, C), jnp.float32),
    )(x, m.reshape(1, -1))
    return out[0]


def _mv(x, stats, count, Sb):
    mean = stats[0] / count
    var = _var(x, mean, Sb) / count
    return mean, var


def _branch(X, xyzT, centersPad, layers, r2, K, Sb_l1, Sb_layer):
    B = X.shape[0]
    S = centersPad.shape[1]
    W1, b1 = layers[0][0], layers[0][1]
    x, stats = _l1(X, xyzT, centersPad, W1.T, b1, r2, K, Sb_l1)
    count = float(B * S * K)
    for i in range(1, len(layers)):
        m, v = _mv(x, stats, count, Sb_layer)
        x, stats = _layer(x, m, v, layers[i - 1][2], layers[i - 1][3],
                          layers[i][0].T, layers[i][1], Sb_layer)
    m, v = _mv(x, stats, count, Sb_layer)
    return _final(x, m, v, layers[-1][2], layers[-1][3], Sb_layer)


def kernel(pc, params):
    B, _, N = pc.shape
    x0 = pc[:, 0, :]
    y0 = pc[:, 1, :]
    z0 = pc[:, 2, :]
    xyzT1 = pc
    X1 = pc.transpose(0, 2, 1)

    cx1, cy1, cz1 = _fps(x0, y0, z0, 512)
    centersT1 = jnp.stack([cx1, cy1, cz1], axis=-1)
    outs1 = []
    for r, K, layers in zip([0.1, 0.2, 0.4], [32, 64, 128], params['sa1']):
        outs1.append(_branch(X1, xyzT1, centersT1, layers,
                             r * r, K, 8, 32))
    f1 = jnp.concatenate(outs1, axis=-1)

    cx2, cy2, cz2 = _fps(cx1, cy1, cz1, 128)
    centersT2 = jnp.stack([cx2, cy2, cz2], axis=-1)
    xyzT2 = jnp.stack([cx1, cy1, cz1], axis=1)
    X2 = jnp.concatenate([centersT1, f1], axis=-1)
    cenPad2 = jnp.concatenate(
        [centersT2, jnp.zeros((B, 128, 320), jnp.float32)], axis=-1)
    outs2 = []
    for r, K, layers in zip([0.2, 0.4, 0.8], [16, 32, 64], params['sa2']):
        outs2.append(_branch(X2, xyzT2, cenPad2, layers,
                             r * r, K, 8, 64))
    f2 = jnp.concatenate(outs2, axis=-1)

    x = f2.reshape(B, 1, 128, 640)
    m = jnp.zeros((640,), jnp.float32)
    v = jnp.full((640,), 1.0 - 1e-5, jnp.float32)
    one = jnp.ones((640,), jnp.float32)
    zero = jnp.zeros((640,), jnp.float32)
    count = float(B * 128)
    glob = params['glob']
    x, stats = _layer(x, m, v, one, zero, glob[0][0].T, glob[0][1], 1)
    for i in (1, 2):
        m, v = _mv(x, stats, count, 1)
        x, stats = _layer(x, m, v, glob[i - 1][2], glob[i - 1][3],
                          glob[i][0].T, glob[i][1], 1)
    m, v = _mv(x, stats, count, 1)
    pooled = _final(x, m, v, glob[2][2], glob[2][3], 1).reshape(B, -1)

    return _fc(pooled, params['fc'])
